# baseline (speedup 1.0000x reference)
import functools

import jax
import jax.numpy as jnp
from jax import lax
from jax.experimental import pallas as pl
from jax.experimental.pallas import tpu as pltpu
from jax.experimental.pallas import tpu_sc as plsc

_B = 16384
_DX = 2213
_NRATE = 6
_NYEAR = 81
_EMB = 32
_BB = 1024
_NCHUNK = 4
_CW = 640

_NC = 2
_NS = 16
_NW = _NC * _NS
_BPW = _B // _NW


def _tc_body(*refs):
    x_refs = refs[:_NCHUNK]
    w_refs = refs[_NCHUNK:2 * _NCHUNK]
    proj_ref, idxr_ref, idxy_ref = refs[2 * _NCHUNK:]

    pre = jnp.zeros((_BB, 2 * _EMB), jnp.float32)
    for xr, wr in zip(x_refs, w_refs):
        xf = xr[...].astype(jnp.bfloat16)
        pre = pre + jnp.dot(xf, wr[...], preferred_element_type=jnp.float32)
    proj = jax.nn.sigmoid(pre)
    proj_ref[...] = jnp.concatenate(
        [jnp.zeros((_BB, 2 * _EMB), jnp.float32), proj], axis=1)

    x01 = x_refs[0][...]
    idxr_ref[...] = x01[:, 0]
    idxy_ref[...] = x01[:, 1]


def _tc_build(interpret=False):
    x_specs = [
        pl.BlockSpec((_BB, _CW), functools.partial(lambda j, i: (i, j), j))
        for j in range(_NCHUNK)
    ]
    w_specs = [pl.BlockSpec((_CW, 2 * _EMB), lambda i: (0, 0)) for _ in range(_NCHUNK)]
    return pl.pallas_call(
        _tc_body,
        grid=(_B // _BB,),
        in_specs=x_specs + w_specs,
        out_specs=[
            pl.BlockSpec((_BB, 4 * _EMB), lambda i: (i, 0)),
            pl.BlockSpec((_BB,), lambda i: (i,)),
            pl.BlockSpec((_BB,), lambda i: (i,)),
        ],
        out_shape=[
            jax.ShapeDtypeStruct((_B, 4 * _EMB), jnp.float32),
            jax.ShapeDtypeStruct((_B,), jnp.int32),
            jax.ShapeDtypeStruct((_B,), jnp.int32),
        ],
        interpret=interpret,
    )


_ROWW = 4 * _EMB


def _sc_body(idxr_hbm, idxy_hbm, rate_hbm, year_hbm, proj_hbm, out_hbm,
             ridx_v, yidx_v, rate_v, year_v, rows_v):
    wid = lax.axis_index("s") * _NC + lax.axis_index("c")
    base = wid * _BPW
    pltpu.sync_copy(idxr_hbm.at[pl.ds(base, _BPW)], ridx_v)
    pltpu.sync_copy(idxy_hbm.at[pl.ds(base, _BPW)], yidx_v)
    pltpu.sync_copy(rate_hbm, rate_v)
    pltpu.sync_copy(year_hbm, year_v)
    pltpu.sync_copy(proj_hbm.at[pl.ds(base * _ROWW, _BPW * _ROWW)], rows_v)

    def group(g, carry):
        rowoff = (lax.iota(jnp.int32, 16) + g * 16) * _ROWW
        ribase = ridx_v[pl.ds(g * 16, 16)] * _ROWW
        yibase = yidx_v[pl.ds(g * 16, 16)] * _ROWW
        for c in range(_EMB):
            rv = plsc.load_gather(rate_v, [ribase + c])
            plsc.store_scatter(rows_v, [rowoff + c], rv)
            yv = plsc.load_gather(year_v, [yibase + c])
            plsc.store_scatter(rows_v, [rowoff + (_EMB + c)], yv)
        return carry

    lax.fori_loop(0, _BPW // 16, group, 0)
    pltpu.sync_copy(rows_v, out_hbm.at[pl.ds(base * _ROWW, _BPW * _ROWW)])


def _sc_build():
    mesh = plsc.VectorSubcoreMesh(core_axis_name="c", subcore_axis_name="s")
    return pl.kernel(
        _sc_body,
        out_type=jax.ShapeDtypeStruct((_B * _ROWW,), jnp.float32),
        mesh=mesh,
        compiler_params=pltpu.CompilerParams(needs_layout_passes=False),
        scratch_types=[
            pltpu.VMEM((_BPW,), jnp.int32),
            pltpu.VMEM((_BPW,), jnp.int32),
            pltpu.VMEM((8 * _ROWW,), jnp.float32),
            pltpu.VMEM((88 * _ROWW,), jnp.float32),
            pltpu.VMEM((_BPW * _ROWW,), jnp.float32),
        ],
    )


def kernel(x, rate_table, year_table, W_genre, W_director):
    wbig = jnp.zeros((_NCHUNK * _CW, 2 * _EMB), jnp.float32)
    wbig = wbig.at[2:27, 0:_EMB].set(W_genre.T)
    wbig = wbig.at[27:_DX, _EMB:].set(W_director.T)
    wbig = wbig.astype(jnp.bfloat16)
    wchunks = [wbig[j * _CW:(j + 1) * _CW] for j in range(_NCHUNK)]
    proj, idxr, idxy = _tc_build()(*([x] * _NCHUNK), *wchunks)
    proj1 = proj.reshape(_B * _ROWW)
    rate_pad = jnp.zeros((8, _ROWW), jnp.float32).at[:_NRATE, :_EMB].set(rate_table)
    year_pad = jnp.zeros((88, _ROWW), jnp.float32).at[:_NYEAR, :_EMB].set(year_table)
    out1 = _sc_build()(idxr, idxy, rate_pad.reshape(-1), year_pad.reshape(-1), proj1)
    return out1.reshape(_B, 4 * _EMB)


# speedup vs baseline: 1.3379x; 1.3379x over previous
import functools

import jax
import jax.numpy as jnp
from jax import lax
from jax.experimental import pallas as pl

_B = 16384
_DX = 2213
_NRATE = 6
_NYEAR = 81
_EMB = 32
_BB = 2048
_NCHUNK = 4
_CW = 640


def _body(*refs):
    x_refs = refs[:_NCHUNK]
    w_refs = refs[_NCHUNK:2 * _NCHUNK]
    rate_ref, year_ref, out_ref = refs[2 * _NCHUNK:]

    pre = jnp.zeros((_BB, 2 * _EMB), jnp.float32)
    for xr, wr in zip(x_refs, w_refs):
        xf = xr[...].astype(jnp.bfloat16)
        pre = pre + jnp.dot(xf, wr[...], preferred_element_type=jnp.float32)
    proj = jax.nn.sigmoid(pre)

    x01 = x_refs[0][...]
    oh_rate = (x01[:, 0:1] == lax.broadcasted_iota(jnp.int32, (_BB, _NRATE), 1)
               ).astype(jnp.float32)
    oh_year = (x01[:, 1:2] == lax.broadcasted_iota(jnp.int32, (_BB, _NYEAR), 1)
               ).astype(jnp.float32)
    rate_emb = jnp.dot(oh_rate, rate_ref[...], preferred_element_type=jnp.float32)
    year_emb = jnp.dot(oh_year, year_ref[...], preferred_element_type=jnp.float32)

    out_ref[...] = jnp.concatenate([rate_emb, year_emb, proj], axis=1)


def _build(interpret=False):
    x_specs = [
        pl.BlockSpec((_BB, _CW), functools.partial(lambda j, i: (i, j), j))
        for j in range(_NCHUNK)
    ]
    w_specs = [pl.BlockSpec((_CW, 2 * _EMB), lambda i: (0, 0)) for _ in range(_NCHUNK)]
    return pl.pallas_call(
        _body,
        grid=(_B // _BB,),
        in_specs=x_specs + w_specs + [
            pl.BlockSpec((_NRATE, _EMB), lambda i: (0, 0)),
            pl.BlockSpec((_NYEAR, _EMB), lambda i: (0, 0)),
        ],
        out_specs=pl.BlockSpec((_BB, 4 * _EMB), lambda i: (i, 0)),
        out_shape=jax.ShapeDtypeStruct((_B, 4 * _EMB), jnp.float32),
        interpret=interpret,
    )


def kernel(x, rate_table, year_table, W_genre, W_director):
    wbig = jnp.zeros((_NCHUNK * _CW, 2 * _EMB), jnp.float32)
    wbig = wbig.at[2:27, 0:_EMB].set(W_genre.T)
    wbig = wbig.at[27:_DX, _EMB:].set(W_director.T)
    wbig = wbig.astype(jnp.bfloat16)
    wchunks = [wbig[j * _CW:(j + 1) * _CW] for j in range(_NCHUNK)]
    return _build()(*([x] * _NCHUNK), *wchunks, rate_table, year_table)


# speedup vs baseline: 1.3487x; 1.0081x over previous
import functools

import jax
import jax.numpy as jnp
from jax import lax
from jax.experimental import pallas as pl

_B = 16384
_DX = 2213
_NRATE = 6
_NYEAR = 81
_EMB = 32
_BB = 1024
_NCHUNK = 6
_CW = 384


def _body(*refs):
    x_refs = refs[:_NCHUNK]
    w_refs = refs[_NCHUNK:2 * _NCHUNK]
    rate_ref, year_ref, out_ref = refs[2 * _NCHUNK:]

    pre = jnp.zeros((_BB, 2 * _EMB), jnp.float32)
    for xr, wr in zip(x_refs, w_refs):
        xf = xr[...].astype(jnp.bfloat16)
        pre = pre + jnp.dot(xf, wr[...], preferred_element_type=jnp.float32)
    proj = jax.nn.sigmoid(pre)

    x01 = x_refs[0][...]
    oh_rate = (x01[:, 0:1] == lax.broadcasted_iota(jnp.int32, (_BB, _NRATE), 1)
               ).astype(jnp.float32)
    oh_year = (x01[:, 1:2] == lax.broadcasted_iota(jnp.int32, (_BB, _NYEAR), 1)
               ).astype(jnp.float32)
    rate_emb = jnp.dot(oh_rate, rate_ref[...], preferred_element_type=jnp.float32)
    year_emb = jnp.dot(oh_year, year_ref[...], preferred_element_type=jnp.float32)

    out_ref[...] = jnp.concatenate([rate_emb, year_emb, proj], axis=1)


def _build(interpret=False):
    x_specs = [
        pl.BlockSpec((_BB, _CW), functools.partial(lambda j, i: (i, j), j))
        for j in range(_NCHUNK)
    ]
    w_specs = [pl.BlockSpec((_CW, 2 * _EMB), lambda i: (0, 0)) for _ in range(_NCHUNK)]
    return pl.pallas_call(
        _body,
        grid=(_B // _BB,),
        in_specs=x_specs + w_specs + [
            pl.BlockSpec((_NRATE, _EMB), lambda i: (0, 0)),
            pl.BlockSpec((_NYEAR, _EMB), lambda i: (0, 0)),
        ],
        out_specs=pl.BlockSpec((_BB, 4 * _EMB), lambda i: (i, 0)),
        out_shape=jax.ShapeDtypeStruct((_B, 4 * _EMB), jnp.float32),
        interpret=interpret,
    )


def kernel(x, rate_table, year_table, W_genre, W_director):
    wbig = jnp.zeros((_NCHUNK * _CW, 2 * _EMB), jnp.float32)
    wbig = wbig.at[2:27, 0:_EMB].set(W_genre.T)
    wbig = wbig.at[27:_DX, _EMB:].set(W_director.T)
    wbig = wbig.astype(jnp.bfloat16)
    wchunks = [wbig[j * _CW:(j + 1) * _CW] for j in range(_NCHUNK)]
    return _build()(*([x] * _NCHUNK), *wchunks, rate_table, year_table)
